# baseline (speedup 1.0000x reference)
import functools

import jax
import jax.numpy as jnp
import numpy as np
from jax import lax
from jax.experimental import pallas as pl
from jax.experimental.pallas import tpu as pltpu
from jax.experimental.pallas import tpu_sc as plsc

_N = 8192
_MAXT = np.array([3.0, 3.0, 2.0], dtype=np.float32)
_NW = 32


def _relu(x):
    return jnp.maximum(x, 0.0)


def _fps_body(b, npoint, n, xg_ref, xr_ref, out_ref, *scratch):
    nb = n // 128
    npb = npoint // 128
    dists = scratch[:b]
    idxs = scratch[b:]
    riota = lax.broadcasted_iota(jnp.int32, (nb, 1), 0)
    liota = lax.broadcasted_iota(jnp.int32, (1, 128), 1)
    iota_p = (lax.broadcasted_iota(jnp.int32, (npb, 128), 0) * 128
              + lax.broadcasted_iota(jnp.int32, (npb, 128), 1))
    for bi in range(b):
        dists[bi][...] = jnp.full((nb, 128), 1e10, jnp.float32)
        idxs[bi][...] = jnp.zeros((npb, 128), jnp.int32)

    def body(i, carry):
        out = []
        for bi in range(b):
            last = carry[bi]
            row = xr_ref[bi, pl.ds(last, 1), :]
            d = ((xg_ref[bi, 0] - row[:, 0:1]) ** 2
                 + (xg_ref[bi, 1] - row[:, 1:2]) ** 2
                 + (xg_ref[bi, 2] - row[:, 2:3]) ** 2)
            nd = jnp.minimum(dists[bi][...], d)
            dists[bi][...] = nd
            rm = jnp.max(nd, axis=1, keepdims=True)
            m = jnp.max(rm)
            r = jnp.min(jnp.where(rm == m, riota, nb))
            drow = dists[bi][pl.ds(r, 1), :]
            lane = jnp.min(jnp.where(drow == m, liota, 128))
            nxt = r * 128 + lane
            idxs[bi][...] = jnp.where(iota_p == i, nxt, idxs[bi][...])
            out.append(nxt)
        return tuple(out)

    lax.fori_loop(1, npoint, body, (jnp.int32(0),) * b)
    for bi in range(b):
        out_ref[bi] = idxs[bi][...]


def _make_fps(b, n, npoint):
    npb = npoint // 128
    return pl.pallas_call(
        functools.partial(_fps_body, b, npoint, n),
        in_specs=[pl.BlockSpec(memory_space=pltpu.VMEM),
                  pl.BlockSpec(memory_space=pltpu.VMEM)],
        out_specs=pl.BlockSpec(memory_space=pltpu.VMEM),
        out_shape=jax.ShapeDtypeStruct((b, npb, 128), jnp.int32),
        scratch_shapes=([pltpu.VMEM((n // 128, 128), jnp.float32)] * b
                        + [pltpu.VMEM((npb, 128), jnp.int32)] * b),
    )


def _ballq_body(n, mc, ns, r2, cen_ref, ptT_ref, out_ref):
    c = cen_ref[0]
    pt = ptT_ref[0]
    ab = jnp.dot(c, pt, preferred_element_type=jnp.float32)
    cn = jnp.sum(c * c, axis=1, keepdims=True)
    pn = jnp.sum(pt * pt, axis=0, keepdims=True)
    d2 = (cn + pn) - 2.0 * ab
    iota = lax.broadcasted_iota(jnp.int32, (mc, n), 1).astype(jnp.float32)
    key = jnp.where(d2 < r2, iota, float(n))
    cols = lax.broadcasted_iota(jnp.int32, (mc, ns), 1)
    g = jnp.full((mc, ns), n, jnp.int32)
    for k in range(ns):
        m = jnp.min(key, axis=1, keepdims=True)
        g = jnp.where(cols == k, m.astype(jnp.int32), g)
        key = jnp.where(key == m, float(n), key)
    first = jnp.broadcast_to(g[:, 0:1], (mc, ns))
    g = jnp.where(g == n, first, g)
    g = jnp.where(g == n, 0, g)
    out_ref[0] = g


def _make_ballq(b, n, m, mc, ns, radius):
    r2 = float(np.float32(radius * radius))
    return pl.pallas_call(
        functools.partial(_ballq_body, n, mc, ns, r2),
        grid=(b, m // mc),
        in_specs=[pl.BlockSpec((1, mc, 8), lambda i, j: (i, j, 0)),
                  pl.BlockSpec((1, 8, n), lambda i, j: (i, 0, 0))],
        out_specs=pl.BlockSpec((1, mc, ns), lambda i, j: (i, j, 0)),
        out_shape=jax.ShapeDtypeStruct((b, m, ns), jnp.int32),
    )


def _mlp_body(mc, ns, g_ref, c_ref, w1_ref, w2_ref, w3_ref, wa_ref, out_ref):
    din = g_ref.shape[2]
    g = g_ref[0].reshape(mc, ns, din)
    gc = (g - c_ref[0][:, None, :]).reshape(mc * ns, din)
    h1 = _relu(jnp.dot(gc, w1_ref[...], preferred_element_type=jnp.float32))
    h2 = _relu(jnp.dot(h1, w2_ref[...], preferred_element_type=jnp.float32))
    h3 = _relu(jnp.dot(h2, w3_ref[...], preferred_element_type=jnp.float32))
    d3 = h3.shape[1]
    mx = jnp.max(h3.reshape(mc, ns, d3), axis=1)
    out_ref[0] = _relu(jnp.dot(mx, wa_ref[...], preferred_element_type=jnp.float32))


def _make_mlp(b, m, mc, ns, din, d1, d2, d3, do):
    return pl.pallas_call(
        functools.partial(_mlp_body, mc, ns),
        grid=(b, m // mc),
        in_specs=[pl.BlockSpec((1, mc * ns, din), lambda i, j: (i, j, 0)),
                  pl.BlockSpec((1, mc, din), lambda i, j: (i, j, 0)),
                  pl.BlockSpec((din, d1), lambda i, j: (0, 0)),
                  pl.BlockSpec((d1, d2), lambda i, j: (0, 0)),
                  pl.BlockSpec((d2, d3), lambda i, j: (0, 0)),
                  pl.BlockSpec((d3, do), lambda i, j: (0, 0))],
        out_specs=pl.BlockSpec((1, mc, do), lambda i, j: (i, j, 0)),
        out_shape=jax.ShapeDtypeStruct((b, m, do), jnp.float32),
    )


def _seg_body(m, kk, f_ref, w1_ref, w2_ref, b2_ref, s_ref, fg_ref, key_ref):
    h = _relu(jnp.dot(f_ref[0], w1_ref[...], preferred_element_type=jnp.float32))
    s = jnp.dot(h, w2_ref[...], preferred_element_type=jnp.float32) + b2_ref[...]
    s_ref[0] = s
    mb = m // 128
    kb = kk // 128
    col = lax.broadcasted_iota(jnp.int32, (m, 8), 1)
    smax = jnp.max(jnp.where(col < 3, s, -1e30), axis=1)
    key_ref[...] = jax.nn.sigmoid(smax).reshape(mb, 128)
    iota_m = (lax.broadcasted_iota(jnp.int32, (mb, 128), 0) * 128
              + lax.broadcasted_iota(jnp.int32, (mb, 128), 1)).astype(jnp.float32)
    iota_k = (lax.broadcasted_iota(jnp.int32, (kb, 128), 0) * 128
              + lax.broadcasted_iota(jnp.int32, (kb, 128), 1)).astype(jnp.float32)
    fg_ref[0] = jnp.zeros((kb, 128), jnp.int32)

    def body(i, _):
        kv = key_ref[...]
        mx = jnp.max(kv)
        pos = jnp.min(jnp.where(kv == mx, iota_m, float(m)))
        fg_ref[0] = jnp.where(iota_k == i.astype(jnp.float32),
                              pos.astype(jnp.int32), fg_ref[0])
        key_ref[...] = jnp.where(iota_m == pos, -3e30, kv)
        return 0

    lax.fori_loop(0, kk, body, 0)


def _make_seg(b, m, kk):
    return pl.pallas_call(
        functools.partial(_seg_body, m, kk),
        grid=(b,),
        in_specs=[pl.BlockSpec((1, m, 128), lambda i: (i, 0, 0)),
                  pl.BlockSpec((128, 64), lambda i: (0, 0)),
                  pl.BlockSpec((64, 8), lambda i: (0, 0)),
                  pl.BlockSpec((1, 8), lambda i: (0, 0))],
        out_specs=[pl.BlockSpec((1, m, 8), lambda i: (i, 0, 0)),
                   pl.BlockSpec((1, kk // 128, 128), lambda i: (i, 0, 0))],
        out_shape=[jax.ShapeDtypeStruct((b, m, 8), jnp.float32),
                   jax.ShapeDtypeStruct((b, kk // 128, 128), jnp.int32)],
        scratch_shapes=[pltpu.VMEM((m // 128, 128), jnp.float32)],
    )


def _vote_body(x3_ref, f3_ref, wm_ref, rw_ref, rb_ref, mt_ref,
               off_ref, x4_ref):
    f3 = f3_ref[0]
    x3 = x3_ref[0]
    vh = _relu(jnp.dot(f3, wm_ref[...], preferred_element_type=jnp.float32))
    off = jnp.dot(vh, rw_ref[...], preferred_element_type=jnp.float32) + rb_ref[...]
    mt = mt_ref[...]
    lim = jnp.clip(off, -mt, mt)
    off_ref[0] = off
    x4_ref[0] = x3 + lim


def _make_vote(b, m):
    return pl.pallas_call(
        _vote_body,
        grid=(b,),
        in_specs=[pl.BlockSpec((1, m, 8), lambda i: (i, 0, 0)),
                  pl.BlockSpec((1, m, 128), lambda i: (i, 0, 0)),
                  pl.BlockSpec((128, 128), lambda i: (0, 0)),
                  pl.BlockSpec((128, 8), lambda i: (0, 0)),
                  pl.BlockSpec((1, 8), lambda i: (0, 0)),
                  pl.BlockSpec((1, 8), lambda i: (0, 0))],
        out_specs=[pl.BlockSpec((1, m, 8), lambda i: (i, 0, 0)),
                   pl.BlockSpec((1, m, 8), lambda i: (i, 0, 0))],
        out_shape=[jax.ShapeDtypeStruct((b, m, 8), jnp.float32),
                   jax.ShapeDtypeStruct((b, m, 8), jnp.float32)],
    )


def _make_gather(b, r, d, mtot):
    cpw = mtot // _NW
    sub = min(cpw, 128)
    nsub = cpw // sub
    wpb = _NW // b
    mesh = plsc.VectorSubcoreMesh(core_axis_name="c", subcore_axis_name="s")

    @functools.partial(
        pl.kernel,
        out_type=jax.ShapeDtypeStruct((mtot, d), jnp.float32),
        mesh=mesh,
        scratch_types=[pltpu.VMEM((sub,), jnp.int32),
                       pltpu.VMEM((sub, d), jnp.float32),
                       pltpu.SemaphoreType.DMA],
    )
    def gk(table_hbm, idx_hbm, out_hbm, idx_v, rows_v, sem):
        wid = lax.axis_index("s") * 2 + lax.axis_index("c")
        base0 = wid * cpw
        boff = (wid // wpb) * r

        def body(j, _):
            base = base0 + j * sub
            pltpu.sync_copy(idx_hbm.at[pl.ds(base, sub)], idx_v)
            for t in range(sub // 16):
                idx_v[pl.ds(t * 16, 16)] = idx_v[pl.ds(t * 16, 16)] + boff
            pltpu.async_copy(table_hbm.at[idx_v], rows_v, sem).wait()
            pltpu.sync_copy(rows_v, out_hbm.at[pl.ds(base, sub)])
            return 0

        lax.fori_loop(0, nsub, body, 0)

    return gk


def _pad3(x, w=8):
    return jnp.concatenate(
        [x, jnp.zeros(x.shape[:-1] + (w - x.shape[-1],), x.dtype)], axis=-1)


def kernel(points, params, batch_size):
    b = points.shape[0] // _N
    n = _N
    xyz = points[:, 1:4].reshape(b, n, 3)
    feat = points[:, 4:5].reshape(b, n, 1)
    bcol = points[:, 0].reshape(b, n)

    p0 = params["sa0"]
    p1 = params["sa1"]
    p4 = params["sa4"]
    w1a = jnp.zeros((8, 32), jnp.float32).at[0:4].set(p0["mlp"][0])
    w1b = jnp.zeros((72, 64), jnp.float32).at[0:67].set(p1["mlp"][0])
    w1c = jnp.zeros((136, 128), jnp.float32).at[0:131].set(p4["mlp"][0])
    segw2 = jnp.zeros((64, 8), jnp.float32).at[:, 0:3].set(params["seg"]["w2"])
    segb2 = jnp.zeros((1, 8), jnp.float32).at[0, 0:3].set(params["seg"]["b2"])
    regw = jnp.zeros((128, 8), jnp.float32).at[:, 0:3].set(params["vote"]["reg_w"])
    regb = jnp.zeros((1, 8), jnp.float32).at[0, 0:3].set(params["vote"]["reg_b"])
    mt = jnp.asarray(np.concatenate([_MAXT, np.full((5,), 1e30, np.float32)])[None])

    xg0 = xyz.transpose(0, 2, 1).reshape(b, 3, n // 128, 128)
    xyz8 = _pad3(xyz)
    idx0 = _make_fps(b, n, 2048)(xg0, xyz8).reshape(b * 2048)
    xyzf128 = _pad3(jnp.concatenate([xyz, feat], axis=-1), 128).reshape(b * n, 128)
    cen0 = _make_gather(b, n, 128, b * 2048)(xyzf128, idx0)[:, 0:3].reshape(b, 2048, 3)
    cen0_8 = _pad3(cen0)
    ptT0 = _pad3(xyz).transpose(0, 2, 1)
    gidx0 = _make_ballq(b, n, 2048, 128, 32, 1.0)(cen0_8, ptT0)
    g0 = _make_gather(b, n, 128, b * 2048 * 32)(
        xyzf128, gidx0.reshape(-1))[:, 0:8].reshape(b, 2048 * 32, 8)
    f1 = _make_mlp(b, 2048, 128, 32, 8, 32, 32, 64, 64)(
        g0, cen0_8, w1a, p0["mlp"][1], p0["mlp"][2], p0["agg"])

    xg1 = cen0.transpose(0, 2, 1).reshape(b, 3, 16, 128)
    idx1 = _make_fps(b, 2048, 512)(xg1, cen0_8).reshape(b * 512)
    in1_128 = _pad3(jnp.concatenate([cen0, f1], axis=-1), 128).reshape(b * 2048, 128)
    cen1 = _make_gather(b, 2048, 128, b * 512)(in1_128, idx1)[:, 0:3].reshape(b, 512, 3)
    cen1_8 = _pad3(cen1)
    ptT1 = _pad3(cen0).transpose(0, 2, 1)
    gidx1 = _make_ballq(b, 2048, 512, 128, 32, 2.0)(cen1_8, ptT1)
    g1 = _make_gather(b, 2048, 128, b * 512 * 32)(
        in1_128, gidx1.reshape(-1))[:, 0:72].reshape(b, 512 * 32, 72)
    cen1_72 = _pad3(cen1, 72)
    f2 = _make_mlp(b, 512, 128, 32, 72, 64, 64, 128, 128)(
        g1, cen1_72, w1b, p1["mlp"][1], p1["mlp"][2], p1["agg"])

    segs, fg = _make_seg(b, 512, 256)(f2, params["seg"]["w1"], segw2, segb2)
    fg_idx = fg.reshape(b * 256)
    t3 = jnp.concatenate([cen1, f2, jnp.zeros((b, 512, 125), jnp.float32)],
                         axis=-1).reshape(b * 512, 256)
    sel = _make_gather(b, 512, 256, b * 256)(t3, fg_idx)
    x3 = sel[:, 0:3].reshape(b, 256, 3)
    f3 = sel[:, 3:131].reshape(b, 256, 128)

    x3_8 = _pad3(x3)
    off8, x4_8 = _make_vote(b, 256)(
        x3_8, f3, params["vote"]["mlp"], regw, regb, mt)
    ptT4 = _pad3(x3).transpose(0, 2, 1)
    gidx4 = _make_ballq(b, 256, 256, 256, 32, 4.8)(x4_8, ptT4)
    t4 = jnp.concatenate([x3, f3, jnp.zeros((b, 256, 125), jnp.float32)],
                         axis=-1).reshape(b * 256, 256)
    g4 = _make_gather(b, 256, 256, b * 256 * 32)(
        t4, gidx4.reshape(-1))[:, 0:136].reshape(b, 256 * 32, 136)
    x4_136 = _pad3(x4_8[..., 0:3], 136)
    f5 = _make_mlp(b, 256, 256, 32, 136, 128, 128, 256, 256)(
        g4, x4_136, w1c, p4["mlp"][1], p4["mlp"][2], p4["agg"])

    ctr_bidx = bcol[:, :256].reshape(-1, 1)
    ctr_offsets_out = jnp.concatenate([ctr_bidx, off8[..., 0:3].reshape(-1, 3)], axis=1)
    centers_out = jnp.concatenate([ctr_bidx, x4_8[..., 0:3].reshape(-1, 3)], axis=1)
    centers_origin_out = jnp.concatenate([ctr_bidx, x3.reshape(-1, 3)], axis=1)
    centers_features = f5.reshape(-1, 256)
    seg_preds = jnp.concatenate([bcol[:, :512][..., None], segs[..., 0:3]], axis=-1)
    seg_points = jnp.concatenate([bcol[:, :512].reshape(-1, 1),
                                  cen1.reshape(-1, 3)], axis=1)
    return (ctr_offsets_out, centers_out, centers_origin_out, centers_features,
            seg_preds, seg_points)


# speedup vs baseline: 1.0527x; 1.0527x over previous
import functools

import jax
import jax.numpy as jnp
import numpy as np
from jax import lax
from jax.experimental import pallas as pl
from jax.experimental.pallas import tpu as pltpu
from jax.experimental.pallas import tpu_sc as plsc

_N = 8192
_MAXT = np.array([3.0, 3.0, 2.0], dtype=np.float32)
_NW = 32


def _relu(x):
    return jnp.maximum(x, 0.0)


def _fps_body(npoint, n, xg_ref, xr_ref, out_ref, dists_ref, idxs_ref):
    nb = n // 128
    npb = npoint // 128
    iota_n = (lax.broadcasted_iota(jnp.int32, (nb, 128), 0) * 128
              + lax.broadcasted_iota(jnp.int32, (nb, 128), 1))
    iota_p = (lax.broadcasted_iota(jnp.int32, (npb, 128), 0) * 128
              + lax.broadcasted_iota(jnp.int32, (npb, 128), 1))
    dists_ref[...] = jnp.full((nb, 128), 1e10, jnp.float32)
    idxs_ref[...] = jnp.zeros((npb, 128), jnp.int32)

    def body(i, last):
        row = xr_ref[0, pl.ds(last, 1), :]
        d = ((xg_ref[0, 0] - row[:, 0:1]) ** 2
             + (xg_ref[0, 1] - row[:, 1:2]) ** 2
             + (xg_ref[0, 2] - row[:, 2:3]) ** 2)
        nd = jnp.minimum(dists_ref[...], d)
        dists_ref[...] = nd
        m = jnp.max(nd)
        nxt = jnp.min(jnp.where(nd == m, iota_n, n))
        idxs_ref[...] = jnp.where(iota_p == i, nxt, idxs_ref[...])
        return nxt

    lax.fori_loop(1, npoint, body, jnp.int32(0))
    out_ref[0] = idxs_ref[...]


def _make_fps(b, n, npoint):
    nb, npb = n // 128, npoint // 128
    return pl.pallas_call(
        functools.partial(_fps_body, npoint, n),
        grid=(b,),
        in_specs=[pl.BlockSpec((1, 3, nb, 128), lambda i: (i, 0, 0, 0)),
                  pl.BlockSpec((1, n, 8), lambda i: (i, 0, 0))],
        out_specs=pl.BlockSpec((1, npb, 128), lambda i: (i, 0, 0)),
        out_shape=jax.ShapeDtypeStruct((b, npb, 128), jnp.int32),
        scratch_shapes=[pltpu.VMEM((nb, 128), jnp.float32),
                        pltpu.VMEM((npb, 128), jnp.int32)],
    )


def _ballq_body(n, mc, ns, r2, cen_ref, ptT_ref, pk_ref, out_ref):
    nw = n // 16
    c = cen_ref[0]
    pt = ptT_ref[0]
    ab = jnp.dot(c, pt, preferred_element_type=jnp.float32)
    cn = jnp.sum(c * c, axis=1, keepdims=True)
    pn = jnp.sum(pt * pt, axis=0, keepdims=True)
    d2 = (cn + pn) - 2.0 * ab
    mask = jnp.where(d2 < r2, jnp.float32(1), jnp.float32(0)).astype(jnp.bfloat16)
    w = jnp.dot(mask, pk_ref[...], preferred_element_type=jnp.float32
                ).astype(jnp.int32)
    lane = lax.broadcasted_iota(jnp.int32, (mc, nw), 1)
    cols = lax.broadcasted_iota(jnp.int32, (mc, ns), 1)
    g = jnp.full((mc, ns), n, jnp.int32)
    for k in range(ns):
        wid = jnp.min(jnp.where(w > 0, lane, nw), axis=1, keepdims=True)
        hit = lane == wid
        wv = jnp.max(jnp.where(hit, w, 0), axis=1, keepdims=True)
        lsb = wv & (-wv)
        t = (lax.bitcast_convert_type(lsb.astype(jnp.float32), jnp.int32)
             >> 23) - 127
        idxk = jnp.where(wv == 0, n, wid * 16 + t)
        g = jnp.where(cols == k, idxk, g)
        w = jnp.where(hit, w & (w - 1), w)
    first = jnp.broadcast_to(g[:, 0:1], (mc, ns))
    g = jnp.where(g == n, first, g)
    g = jnp.where(g == n, 0, g)
    out_ref[0] = g


def _make_ballq(b, n, m, mc, ns, radius):
    r2 = float(np.float32(radius * radius))
    nw = n // 16
    call = pl.pallas_call(
        functools.partial(_ballq_body, n, mc, ns, r2),
        grid=(b, m // mc),
        in_specs=[pl.BlockSpec((1, mc, 8), lambda i, j: (i, j, 0)),
                  pl.BlockSpec((1, 8, n), lambda i, j: (i, 0, 0)),
                  pl.BlockSpec((n, nw), lambda i, j: (0, 0))],
        out_specs=pl.BlockSpec((1, mc, ns), lambda i, j: (i, j, 0)),
        out_shape=jax.ShapeDtypeStruct((b, m, ns), jnp.int32),
    )
    pknp = np.zeros((n, nw), np.float32)
    ar = np.arange(n)
    pknp[ar, ar // 16] = (2.0 ** (ar % 16)).astype(np.float32)
    pk = jnp.asarray(pknp, jnp.bfloat16)
    return lambda cen, ptT: call(cen, ptT, pk)


def _mlp_body(mc, ns, g_ref, c_ref, w1_ref, w2_ref, w3_ref, wa_ref, out_ref):
    din = g_ref.shape[2]
    g = g_ref[0].reshape(mc, ns, din)
    gc = (g - c_ref[0][:, None, :]).reshape(mc * ns, din)
    h1 = _relu(jnp.dot(gc, w1_ref[...], preferred_element_type=jnp.float32))
    h2 = _relu(jnp.dot(h1, w2_ref[...], preferred_element_type=jnp.float32))
    h3 = _relu(jnp.dot(h2, w3_ref[...], preferred_element_type=jnp.float32))
    d3 = h3.shape[1]
    mx = jnp.max(h3.reshape(mc, ns, d3), axis=1)
    out_ref[0] = _relu(jnp.dot(mx, wa_ref[...], preferred_element_type=jnp.float32))


def _make_mlp(b, m, mc, ns, din, d1, d2, d3, do):
    return pl.pallas_call(
        functools.partial(_mlp_body, mc, ns),
        grid=(b, m // mc),
        in_specs=[pl.BlockSpec((1, mc * ns, din), lambda i, j: (i, j, 0)),
                  pl.BlockSpec((1, mc, din), lambda i, j: (i, j, 0)),
                  pl.BlockSpec((din, d1), lambda i, j: (0, 0)),
                  pl.BlockSpec((d1, d2), lambda i, j: (0, 0)),
                  pl.BlockSpec((d2, d3), lambda i, j: (0, 0)),
                  pl.BlockSpec((d3, do), lambda i, j: (0, 0))],
        out_specs=pl.BlockSpec((1, mc, do), lambda i, j: (i, j, 0)),
        out_shape=jax.ShapeDtypeStruct((b, m, do), jnp.float32),
    )


def _seg_body(m, kk, f_ref, w1_ref, w2_ref, b2_ref, s_ref, fg_ref, key_ref):
    h = _relu(jnp.dot(f_ref[0], w1_ref[...], preferred_element_type=jnp.float32))
    s = jnp.dot(h, w2_ref[...], preferred_element_type=jnp.float32) + b2_ref[...]
    s_ref[0] = s
    mb = m // 128
    kb = kk // 128
    col = lax.broadcasted_iota(jnp.int32, (m, 8), 1)
    smax = jnp.max(jnp.where(col < 3, s, -1e30), axis=1)
    key_ref[...] = jax.nn.sigmoid(smax).reshape(mb, 128)
    iota_m = (lax.broadcasted_iota(jnp.int32, (mb, 128), 0) * 128
              + lax.broadcasted_iota(jnp.int32, (mb, 128), 1)).astype(jnp.float32)
    iota_k = (lax.broadcasted_iota(jnp.int32, (kb, 128), 0) * 128
              + lax.broadcasted_iota(jnp.int32, (kb, 128), 1)).astype(jnp.float32)
    fg_ref[0] = jnp.zeros((kb, 128), jnp.int32)

    def body(i, _):
        kv = key_ref[...]
        mx = jnp.max(kv)
        pos = jnp.min(jnp.where(kv == mx, iota_m, float(m)))
        fg_ref[0] = jnp.where(iota_k == i.astype(jnp.float32),
                              pos.astype(jnp.int32), fg_ref[0])
        key_ref[...] = jnp.where(iota_m == pos, -3e30, kv)
        return 0

    lax.fori_loop(0, kk, body, 0)


def _make_seg(b, m, kk):
    return pl.pallas_call(
        functools.partial(_seg_body, m, kk),
        grid=(b,),
        in_specs=[pl.BlockSpec((1, m, 128), lambda i: (i, 0, 0)),
                  pl.BlockSpec((128, 64), lambda i: (0, 0)),
                  pl.BlockSpec((64, 8), lambda i: (0, 0)),
                  pl.BlockSpec((1, 8), lambda i: (0, 0))],
        out_specs=[pl.BlockSpec((1, m, 8), lambda i: (i, 0, 0)),
                   pl.BlockSpec((1, kk // 128, 128), lambda i: (i, 0, 0))],
        out_shape=[jax.ShapeDtypeStruct((b, m, 8), jnp.float32),
                   jax.ShapeDtypeStruct((b, kk // 128, 128), jnp.int32)],
        scratch_shapes=[pltpu.VMEM((m // 128, 128), jnp.float32)],
    )


def _vote_body(x3_ref, f3_ref, wm_ref, rw_ref, rb_ref, mt_ref,
               off_ref, x4_ref):
    f3 = f3_ref[0]
    x3 = x3_ref[0]
    vh = _relu(jnp.dot(f3, wm_ref[...], preferred_element_type=jnp.float32))
    off = jnp.dot(vh, rw_ref[...], preferred_element_type=jnp.float32) + rb_ref[...]
    mt = mt_ref[...]
    lim = jnp.clip(off, -mt, mt)
    off_ref[0] = off
    x4_ref[0] = x3 + lim


def _make_vote(b, m):
    return pl.pallas_call(
        _vote_body,
        grid=(b,),
        in_specs=[pl.BlockSpec((1, m, 8), lambda i: (i, 0, 0)),
                  pl.BlockSpec((1, m, 128), lambda i: (i, 0, 0)),
                  pl.BlockSpec((128, 128), lambda i: (0, 0)),
                  pl.BlockSpec((128, 8), lambda i: (0, 0)),
                  pl.BlockSpec((1, 8), lambda i: (0, 0)),
                  pl.BlockSpec((1, 8), lambda i: (0, 0))],
        out_specs=[pl.BlockSpec((1, m, 8), lambda i: (i, 0, 0)),
                   pl.BlockSpec((1, m, 8), lambda i: (i, 0, 0))],
        out_shape=[jax.ShapeDtypeStruct((b, m, 8), jnp.float32),
                   jax.ShapeDtypeStruct((b, m, 8), jnp.float32)],
    )


def _make_gather(b, r, d, mtot):
    cpw = mtot // _NW
    sub = min(cpw, 128)
    nsub = cpw // sub
    wpb = _NW // b
    mesh = plsc.VectorSubcoreMesh(core_axis_name="c", subcore_axis_name="s")

    @functools.partial(
        pl.kernel,
        out_type=jax.ShapeDtypeStruct((mtot, d), jnp.float32),
        mesh=mesh,
        scratch_types=[pltpu.VMEM((sub,), jnp.int32),
                       pltpu.VMEM((sub, d), jnp.float32),
                       pltpu.SemaphoreType.DMA],
    )
    def gk(table_hbm, idx_hbm, out_hbm, idx_v, rows_v, sem):
        wid = lax.axis_index("s") * 2 + lax.axis_index("c")
        base0 = wid * cpw
        boff = (wid // wpb) * r

        def body(j, _):
            base = base0 + j * sub
            pltpu.sync_copy(idx_hbm.at[pl.ds(base, sub)], idx_v)
            for t in range(sub // 16):
                idx_v[pl.ds(t * 16, 16)] = idx_v[pl.ds(t * 16, 16)] + boff
            pltpu.async_copy(table_hbm.at[idx_v], rows_v, sem).wait()
            pltpu.sync_copy(rows_v, out_hbm.at[pl.ds(base, sub)])
            return 0

        lax.fori_loop(0, nsub, body, 0)

    return gk


def _pad3(x, w=8):
    return jnp.concatenate(
        [x, jnp.zeros(x.shape[:-1] + (w - x.shape[-1],), x.dtype)], axis=-1)


def kernel(points, params, batch_size):
    b = points.shape[0] // _N
    n = _N
    xyz = points[:, 1:4].reshape(b, n, 3)
    feat = points[:, 4:5].reshape(b, n, 1)
    bcol = points[:, 0].reshape(b, n)

    p0 = params["sa0"]
    p1 = params["sa1"]
    p4 = params["sa4"]
    w1a = jnp.zeros((8, 32), jnp.float32).at[0:4].set(p0["mlp"][0])
    w1b = jnp.zeros((72, 64), jnp.float32).at[0:67].set(p1["mlp"][0])
    w1c = jnp.zeros((136, 128), jnp.float32).at[0:131].set(p4["mlp"][0])
    segw2 = jnp.zeros((64, 8), jnp.float32).at[:, 0:3].set(params["seg"]["w2"])
    segb2 = jnp.zeros((1, 8), jnp.float32).at[0, 0:3].set(params["seg"]["b2"])
    regw = jnp.zeros((128, 8), jnp.float32).at[:, 0:3].set(params["vote"]["reg_w"])
    regb = jnp.zeros((1, 8), jnp.float32).at[0, 0:3].set(params["vote"]["reg_b"])
    mt = jnp.asarray(np.concatenate([_MAXT, np.full((5,), 1e30, np.float32)])[None])

    xg0 = xyz.transpose(0, 2, 1).reshape(b, 3, n // 128, 128)
    xyz8 = _pad3(xyz)
    idx0 = _make_fps(b, n, 2048)(xg0, xyz8).reshape(b * 2048)
    xyzf128 = _pad3(jnp.concatenate([xyz, feat], axis=-1), 128).reshape(b * n, 128)
    cen0 = _make_gather(b, n, 128, b * 2048)(xyzf128, idx0)[:, 0:3].reshape(b, 2048, 3)
    cen0_8 = _pad3(cen0)
    ptT0 = _pad3(xyz).transpose(0, 2, 1)
    gidx0 = _make_ballq(b, n, 2048, 128, 32, 1.0)(cen0_8, ptT0)
    g0 = _make_gather(b, n, 128, b * 2048 * 32)(
        xyzf128, gidx0.reshape(-1))[:, 0:8].reshape(b, 2048 * 32, 8)
    f1 = _make_mlp(b, 2048, 128, 32, 8, 32, 32, 64, 64)(
        g0, cen0_8, w1a, p0["mlp"][1], p0["mlp"][2], p0["agg"])

    xg1 = cen0.transpose(0, 2, 1).reshape(b, 3, 16, 128)
    idx1 = _make_fps(b, 2048, 512)(xg1, cen0_8).reshape(b * 512)
    in1_128 = _pad3(jnp.concatenate([cen0, f1], axis=-1), 128).reshape(b * 2048, 128)
    cen1 = _make_gather(b, 2048, 128, b * 512)(in1_128, idx1)[:, 0:3].reshape(b, 512, 3)
    cen1_8 = _pad3(cen1)
    ptT1 = _pad3(cen0).transpose(0, 2, 1)
    gidx1 = _make_ballq(b, 2048, 512, 128, 32, 2.0)(cen1_8, ptT1)
    g1 = _make_gather(b, 2048, 128, b * 512 * 32)(
        in1_128, gidx1.reshape(-1))[:, 0:72].reshape(b, 512 * 32, 72)
    cen1_72 = _pad3(cen1, 72)
    f2 = _make_mlp(b, 512, 128, 32, 72, 64, 64, 128, 128)(
        g1, cen1_72, w1b, p1["mlp"][1], p1["mlp"][2], p1["agg"])

    segs, fg = _make_seg(b, 512, 256)(f2, params["seg"]["w1"], segw2, segb2)
    fg_idx = fg.reshape(b * 256)
    t3 = jnp.concatenate([cen1, f2, jnp.zeros((b, 512, 125), jnp.float32)],
                         axis=-1).reshape(b * 512, 256)
    sel = _make_gather(b, 512, 256, b * 256)(t3, fg_idx)
    x3 = sel[:, 0:3].reshape(b, 256, 3)
    f3 = sel[:, 3:131].reshape(b, 256, 128)

    x3_8 = _pad3(x3)
    off8, x4_8 = _make_vote(b, 256)(
        x3_8, f3, params["vote"]["mlp"], regw, regb, mt)
    ptT4 = _pad3(x3).transpose(0, 2, 1)
    gidx4 = _make_ballq(b, 256, 256, 256, 32, 4.8)(x4_8, ptT4)
    t4 = jnp.concatenate([x3, f3, jnp.zeros((b, 256, 125), jnp.float32)],
                         axis=-1).reshape(b * 256, 256)
    g4 = _make_gather(b, 256, 256, b * 256 * 32)(
        t4, gidx4.reshape(-1))[:, 0:136].reshape(b, 256 * 32, 136)
    x4_136 = _pad3(x4_8[..., 0:3], 136)
    f5 = _make_mlp(b, 256, 256, 32, 136, 128, 128, 256, 256)(
        g4, x4_136, w1c, p4["mlp"][1], p4["mlp"][2], p4["agg"])

    ctr_bidx = bcol[:, :256].reshape(-1, 1)
    ctr_offsets_out = jnp.concatenate([ctr_bidx, off8[..., 0:3].reshape(-1, 3)], axis=1)
    centers_out = jnp.concatenate([ctr_bidx, x4_8[..., 0:3].reshape(-1, 3)], axis=1)
    centers_origin_out = jnp.concatenate([ctr_bidx, x3.reshape(-1, 3)], axis=1)
    centers_features = f5.reshape(-1, 256)
    seg_preds = jnp.concatenate([bcol[:, :512][..., None], segs[..., 0:3]], axis=-1)
    seg_points = jnp.concatenate([bcol[:, :512].reshape(-1, 1),
                                  cen1.reshape(-1, 3)], axis=1)
    return (ctr_offsets_out, centers_out, centers_origin_out, centers_features,
            seg_preds, seg_points)


# speedup vs baseline: 1.0905x; 1.0360x over previous
import functools

import jax
import jax.numpy as jnp
import numpy as np
from jax import lax
from jax.experimental import pallas as pl
from jax.experimental.pallas import tpu as pltpu
from jax.experimental.pallas import tpu_sc as plsc

_N = 8192
_MAXT = np.array([3.0, 3.0, 2.0], dtype=np.float32)
_NW = 32


def _relu(x):
    return jnp.maximum(x, 0.0)


def _fps_body(npoint, n, xg_ref, xr_ref, out_ref, dists_ref, idxs_ref):
    nb = n // 128
    npb = npoint // 128
    iota_n = (lax.broadcasted_iota(jnp.int32, (nb, 128), 0) * 128
              + lax.broadcasted_iota(jnp.int32, (nb, 128), 1))
    iota_p = (lax.broadcasted_iota(jnp.int32, (npb, 128), 0) * 128
              + lax.broadcasted_iota(jnp.int32, (npb, 128), 1))
    dists_ref[...] = jnp.full((nb, 128), 1e10, jnp.float32)
    idxs_ref[...] = jnp.zeros((npb, 128), jnp.int32)

    def body(i, last):
        row = xr_ref[0, pl.ds(last, 1), :]
        d = ((xg_ref[0, 0] - row[:, 0:1]) ** 2
             + (xg_ref[0, 1] - row[:, 1:2]) ** 2
             + (xg_ref[0, 2] - row[:, 2:3]) ** 2)
        nd = jnp.minimum(dists_ref[...], d)
        dists_ref[...] = nd
        m = jnp.max(nd)
        nxt = jnp.min(jnp.where(nd == m, iota_n, n))
        idxs_ref[...] = jnp.where(iota_p == i, nxt, idxs_ref[...])
        return nxt

    lax.fori_loop(1, npoint, body, jnp.int32(0))
    out_ref[0] = idxs_ref[...]


def _make_fps(b, n, npoint):
    nb, npb = n // 128, npoint // 128
    return pl.pallas_call(
        functools.partial(_fps_body, npoint, n),
        grid=(b,),
        in_specs=[pl.BlockSpec((1, 3, nb, 128), lambda i: (i, 0, 0, 0)),
                  pl.BlockSpec((1, n, 8), lambda i: (i, 0, 0))],
        out_specs=pl.BlockSpec((1, npb, 128), lambda i: (i, 0, 0)),
        out_shape=jax.ShapeDtypeStruct((b, npb, 128), jnp.int32),
        scratch_shapes=[pltpu.VMEM((nb, 128), jnp.float32),
                        pltpu.VMEM((npb, 128), jnp.int32)],
    )


def _ballq_body(n, mc, ns, r2, cen_ref, ptT_ref, pk_ref, out_ref):
    nw = n // 16
    c = cen_ref[0]
    pt = ptT_ref[0]
    ab = jnp.dot(c, pt, preferred_element_type=jnp.float32)
    cn = jnp.sum(c * c, axis=1, keepdims=True)
    pn = jnp.sum(pt * pt, axis=0, keepdims=True)
    d2 = (cn + pn) - 2.0 * ab
    mask = jnp.where(d2 < r2, jnp.float32(1), jnp.float32(0)).astype(jnp.bfloat16)
    w = jnp.dot(mask, pk_ref[...], preferred_element_type=jnp.float32
                ).astype(jnp.int32)
    lane = lax.broadcasted_iota(jnp.int32, (mc, nw), 1)
    cols = lax.broadcasted_iota(jnp.int32, (mc, ns), 1)
    g = jnp.full((mc, ns), n, jnp.int32)
    for k in range(ns):
        wid = jnp.min(jnp.where(w > 0, lane, nw), axis=1, keepdims=True)
        hit = lane == wid
        wv = jnp.max(jnp.where(hit, w, 0), axis=1, keepdims=True)
        lsb = wv & (-wv)
        t = (lax.bitcast_convert_type(lsb.astype(jnp.float32), jnp.int32)
             >> 23) - 127
        idxk = jnp.where(wv == 0, n, wid * 16 + t)
        g = jnp.where(cols == k, idxk, g)
        w = jnp.where(hit, w & (w - 1), w)
    first = jnp.broadcast_to(g[:, 0:1], (mc, ns))
    g = jnp.where(g == n, first, g)
    g = jnp.where(g == n, 0, g)
    out_ref[0] = g


def _make_ballq(b, n, m, mc, ns, radius):
    r2 = float(np.float32(radius * radius))
    nw = n // 16
    call = pl.pallas_call(
        functools.partial(_ballq_body, n, mc, ns, r2),
        grid=(b, m // mc),
        in_specs=[pl.BlockSpec((1, mc, 8), lambda i, j: (i, j, 0)),
                  pl.BlockSpec((1, 8, n), lambda i, j: (i, 0, 0)),
                  pl.BlockSpec((n, nw), lambda i, j: (0, 0))],
        out_specs=pl.BlockSpec((1, mc, ns), lambda i, j: (i, j, 0)),
        out_shape=jax.ShapeDtypeStruct((b, m, ns), jnp.int32),
    )
    pknp = np.zeros((n, nw), np.float32)
    ar = np.arange(n)
    pknp[ar, ar // 16] = (2.0 ** (ar % 16)).astype(np.float32)
    pk = jnp.asarray(pknp, jnp.bfloat16)
    return lambda cen, ptT: call(cen, ptT, pk)


def _mlp_body(mc, ns, g_ref, c_ref, w1_ref, w2_ref, w3_ref, wa_ref, out_ref):
    din = g_ref.shape[2]
    g = g_ref[0].reshape(mc, ns, din)
    gc = (g - c_ref[0][:, None, :]).reshape(mc * ns, din)
    h1 = _relu(jnp.dot(gc, w1_ref[...], preferred_element_type=jnp.float32))
    h2 = _relu(jnp.dot(h1, w2_ref[...], preferred_element_type=jnp.float32))
    h3 = _relu(jnp.dot(h2, w3_ref[...], preferred_element_type=jnp.float32))
    d3 = h3.shape[1]
    mx = jnp.max(h3.reshape(mc, ns, d3), axis=1)
    out_ref[0] = _relu(jnp.dot(mx, wa_ref[...], preferred_element_type=jnp.float32))


def _make_mlp(b, m, mc, ns, din, d1, d2, d3, do):
    return pl.pallas_call(
        functools.partial(_mlp_body, mc, ns),
        grid=(b, m // mc),
        in_specs=[pl.BlockSpec((1, mc * ns, din), lambda i, j: (i, j, 0)),
                  pl.BlockSpec((1, mc, din), lambda i, j: (i, j, 0)),
                  pl.BlockSpec((din, d1), lambda i, j: (0, 0)),
                  pl.BlockSpec((d1, d2), lambda i, j: (0, 0)),
                  pl.BlockSpec((d2, d3), lambda i, j: (0, 0)),
                  pl.BlockSpec((d3, do), lambda i, j: (0, 0))],
        out_specs=pl.BlockSpec((1, mc, do), lambda i, j: (i, j, 0)),
        out_shape=jax.ShapeDtypeStruct((b, m, do), jnp.float32),
    )


def _seg_body(m, kk, f_ref, w1_ref, w2_ref, b2_ref, s_ref, fg_ref, key_ref):
    h = _relu(jnp.dot(f_ref[0], w1_ref[...], preferred_element_type=jnp.float32))
    s = jnp.dot(h, w2_ref[...], preferred_element_type=jnp.float32) + b2_ref[...]
    s_ref[0] = s
    mb = m // 128
    kb = kk // 128
    col = lax.broadcasted_iota(jnp.int32, (m, 8), 1)
    smax = jnp.max(jnp.where(col < 3, s, -1e30), axis=1)
    key_ref[...] = jax.nn.sigmoid(smax).reshape(mb, 128)
    iota_m = (lax.broadcasted_iota(jnp.int32, (mb, 128), 0) * 128
              + lax.broadcasted_iota(jnp.int32, (mb, 128), 1)).astype(jnp.float32)
    iota_k = (lax.broadcasted_iota(jnp.int32, (kb, 128), 0) * 128
              + lax.broadcasted_iota(jnp.int32, (kb, 128), 1)).astype(jnp.float32)
    fg_ref[0] = jnp.zeros((kb, 128), jnp.int32)

    def body(i, _):
        kv = key_ref[...]
        mx = jnp.max(kv)
        pos = jnp.min(jnp.where(kv == mx, iota_m, float(m)))
        fg_ref[0] = jnp.where(iota_k == i.astype(jnp.float32),
                              pos.astype(jnp.int32), fg_ref[0])
        key_ref[...] = jnp.where(iota_m == pos, -3e30, kv)
        return 0

    lax.fori_loop(0, kk, body, 0)


def _make_seg(b, m, kk):
    return pl.pallas_call(
        functools.partial(_seg_body, m, kk),
        grid=(b,),
        in_specs=[pl.BlockSpec((1, m, 128), lambda i: (i, 0, 0)),
                  pl.BlockSpec((128, 64), lambda i: (0, 0)),
                  pl.BlockSpec((64, 8), lambda i: (0, 0)),
                  pl.BlockSpec((1, 8), lambda i: (0, 0))],
        out_specs=[pl.BlockSpec((1, m, 8), lambda i: (i, 0, 0)),
                   pl.BlockSpec((1, kk // 128, 128), lambda i: (i, 0, 0))],
        out_shape=[jax.ShapeDtypeStruct((b, m, 8), jnp.float32),
                   jax.ShapeDtypeStruct((b, kk // 128, 128), jnp.int32)],
        scratch_shapes=[pltpu.VMEM((m // 128, 128), jnp.float32)],
    )


def _vote_body(x3_ref, f3_ref, wm_ref, rw_ref, rb_ref, mt_ref,
               off_ref, x4_ref):
    f3 = f3_ref[0]
    x3 = x3_ref[0]
    vh = _relu(jnp.dot(f3, wm_ref[...], preferred_element_type=jnp.float32))
    off = jnp.dot(vh, rw_ref[...], preferred_element_type=jnp.float32) + rb_ref[...]
    mt = mt_ref[...]
    lim = jnp.clip(off, -mt, mt)
    off_ref[0] = off
    x4_ref[0] = x3 + lim


def _make_vote(b, m):
    return pl.pallas_call(
        _vote_body,
        grid=(b,),
        in_specs=[pl.BlockSpec((1, m, 8), lambda i: (i, 0, 0)),
                  pl.BlockSpec((1, m, 128), lambda i: (i, 0, 0)),
                  pl.BlockSpec((128, 128), lambda i: (0, 0)),
                  pl.BlockSpec((128, 8), lambda i: (0, 0)),
                  pl.BlockSpec((1, 8), lambda i: (0, 0)),
                  pl.BlockSpec((1, 8), lambda i: (0, 0))],
        out_specs=[pl.BlockSpec((1, m, 8), lambda i: (i, 0, 0)),
                   pl.BlockSpec((1, m, 8), lambda i: (i, 0, 0))],
        out_shape=[jax.ShapeDtypeStruct((b, m, 8), jnp.float32),
                   jax.ShapeDtypeStruct((b, m, 8), jnp.float32)],
    )


def _make_gather(b, r, d, mtot):
    cpw = mtot // _NW
    sub = min(cpw, 128)
    nsub = cpw // sub
    wpb = _NW // b
    mesh = plsc.VectorSubcoreMesh(core_axis_name="c", subcore_axis_name="s")

    @functools.partial(
        pl.kernel,
        out_type=jax.ShapeDtypeStruct((mtot, d), jnp.float32),
        mesh=mesh,
        scratch_types=[pltpu.VMEM((sub,), jnp.int32),
                       pltpu.VMEM((sub, d), jnp.float32),
                       pltpu.SemaphoreType.DMA],
    )
    def gk(table_hbm, idx_hbm, out_hbm, idx_v, rows_v, sem):
        wid = lax.axis_index("s") * 2 + lax.axis_index("c")
        base0 = wid * cpw
        boff = (wid // wpb) * r

        def body(j, _):
            base = base0 + j * sub
            pltpu.sync_copy(idx_hbm.at[pl.ds(base, sub)], idx_v)
            for t in range(sub // 16):
                idx_v[pl.ds(t * 16, 16)] = idx_v[pl.ds(t * 16, 16)] + boff
            pltpu.async_copy(table_hbm.at[idx_v], rows_v, sem).wait()
            pltpu.sync_copy(rows_v, out_hbm.at[pl.ds(base, sub)])
            return 0

        lax.fori_loop(0, nsub, body, 0)

    return gk


def _pad3(x, w=8):
    return jnp.concatenate(
        [x, jnp.zeros(x.shape[:-1] + (w - x.shape[-1],), x.dtype)], axis=-1)


def kernel(points, params, batch_size):
    b = points.shape[0] // _N
    n = _N
    xyz = points[:, 1:4].reshape(b, n, 3)
    feat = points[:, 4:5].reshape(b, n, 1)
    bcol = points[:, 0].reshape(b, n)

    p0 = params["sa0"]
    p1 = params["sa1"]
    p4 = params["sa4"]
    w1a = jnp.zeros((8, 32), jnp.float32).at[0:4].set(p0["mlp"][0])
    w1b = jnp.zeros((72, 64), jnp.float32).at[0:67].set(p1["mlp"][0])
    w1c = jnp.zeros((136, 128), jnp.float32).at[0:131].set(p4["mlp"][0])
    segw2 = jnp.zeros((64, 8), jnp.float32).at[:, 0:3].set(params["seg"]["w2"])
    segb2 = jnp.zeros((1, 8), jnp.float32).at[0, 0:3].set(params["seg"]["b2"])
    regw = jnp.zeros((128, 8), jnp.float32).at[:, 0:3].set(params["vote"]["reg_w"])
    regb = jnp.zeros((1, 8), jnp.float32).at[0, 0:3].set(params["vote"]["reg_b"])
    mt = jnp.asarray(np.concatenate([_MAXT, np.full((5,), 1e30, np.float32)])[None])

    xg0 = xyz.transpose(0, 2, 1).reshape(b, 3, n // 128, 128)
    xyz8 = _pad3(xyz)
    idx0 = _make_fps(b, n, 2048)(xg0, xyz8).reshape(b * 2048)
    xyzf128 = _pad3(jnp.concatenate([xyz, feat], axis=-1), 128).reshape(b * n, 128)
    cen0 = _make_gather(b, n, 128, b * 2048)(xyzf128, idx0)[:, 0:3].reshape(b, 2048, 3)
    cen0_8 = _pad3(cen0)
    ptT0 = _pad3(xyz).transpose(0, 2, 1)
    gidx0 = _make_ballq(b, n, 2048, 512, 32, 1.0)(cen0_8, ptT0)
    g0 = _make_gather(b, n, 128, b * 2048 * 32)(
        xyzf128, gidx0.reshape(-1))[:, 0:8].reshape(b, 2048 * 32, 8)
    f1 = _make_mlp(b, 2048, 128, 32, 8, 32, 32, 64, 64)(
        g0, cen0_8, w1a, p0["mlp"][1], p0["mlp"][2], p0["agg"])

    xg1 = cen0.transpose(0, 2, 1).reshape(b, 3, 16, 128)
    idx1 = _make_fps(b, 2048, 512)(xg1, cen0_8).reshape(b * 512)
    in1_128 = _pad3(jnp.concatenate([cen0, f1], axis=-1), 128).reshape(b * 2048, 128)
    cen1 = _make_gather(b, 2048, 128, b * 512)(in1_128, idx1)[:, 0:3].reshape(b, 512, 3)
    cen1_8 = _pad3(cen1)
    ptT1 = _pad3(cen0).transpose(0, 2, 1)
    gidx1 = _make_ballq(b, 2048, 512, 512, 32, 2.0)(cen1_8, ptT1)
    g1 = _make_gather(b, 2048, 128, b * 512 * 32)(
        in1_128, gidx1.reshape(-1))[:, 0:72].reshape(b, 512 * 32, 72)
    cen1_72 = _pad3(cen1, 72)
    f2 = _make_mlp(b, 512, 128, 32, 72, 64, 64, 128, 128)(
        g1, cen1_72, w1b, p1["mlp"][1], p1["mlp"][2], p1["agg"])

    segs, fg = _make_seg(b, 512, 256)(f2, params["seg"]["w1"], segw2, segb2)
    fg_idx = fg.reshape(b * 256)
    t3 = jnp.concatenate([cen1, f2, jnp.zeros((b, 512, 125), jnp.float32)],
                         axis=-1).reshape(b * 512, 256)
    sel = _make_gather(b, 512, 256, b * 256)(t3, fg_idx)
    x3 = sel[:, 0:3].reshape(b, 256, 3)
    f3 = sel[:, 3:131].reshape(b, 256, 128)

    x3_8 = _pad3(x3)
    off8, x4_8 = _make_vote(b, 256)(
        x3_8, f3, params["vote"]["mlp"], regw, regb, mt)
    ptT4 = _pad3(x3).transpose(0, 2, 1)
    gidx4 = _make_ballq(b, 256, 256, 256, 32, 4.8)(x4_8, ptT4)
    t4 = jnp.concatenate([x3, f3, jnp.zeros((b, 256, 125), jnp.float32)],
                         axis=-1).reshape(b * 256, 256)
    g4 = _make_gather(b, 256, 256, b * 256 * 32)(
        t4, gidx4.reshape(-1))[:, 0:136].reshape(b, 256 * 32, 136)
    x4_136 = _pad3(x4_8[..., 0:3], 136)
    f5 = _make_mlp(b, 256, 256, 32, 136, 128, 128, 256, 256)(
        g4, x4_136, w1c, p4["mlp"][1], p4["mlp"][2], p4["agg"])

    ctr_bidx = bcol[:, :256].reshape(-1, 1)
    ctr_offsets_out = jnp.concatenate([ctr_bidx, off8[..., 0:3].reshape(-1, 3)], axis=1)
    centers_out = jnp.concatenate([ctr_bidx, x4_8[..., 0:3].reshape(-1, 3)], axis=1)
    centers_origin_out = jnp.concatenate([ctr_bidx, x3.reshape(-1, 3)], axis=1)
    centers_features = f5.reshape(-1, 256)
    seg_preds = jnp.concatenate([bcol[:, :512][..., None], segs[..., 0:3]], axis=-1)
    seg_points = jnp.concatenate([bcol[:, :512].reshape(-1, 1),
                                  cen1.reshape(-1, 3)], axis=1)
    return (ctr_offsets_out, centers_out, centers_origin_out, centers_features,
            seg_preds, seg_points)


# speedup vs baseline: 1.6433x; 1.5069x over previous
import functools

import jax
import jax.numpy as jnp
import numpy as np
from jax import lax
from jax.experimental import pallas as pl
from jax.experimental.pallas import tpu as pltpu
from jax.experimental.pallas import tpu_sc as plsc

_N = 8192
_MAXT = np.array([3.0, 3.0, 2.0], dtype=np.float32)
_NW = 32


def _relu(x):
    return jnp.maximum(x, 0.0)


def _fps_body(b, npoint, n, xc_ref, xr_ref, out_ref, dists_ref, idxs_ref):
    nb = n // 128
    npb = npoint // 128
    rr, rp = b * nb, b * npb
    riota = lax.broadcasted_iota(jnp.int32, (rr, 1), 0)
    riota_p = lax.broadcasted_iota(jnp.int32, (rp, 1), 0)
    iota_n = ((lax.broadcasted_iota(jnp.int32, (rr, 128), 0) % nb) * 128
              + lax.broadcasted_iota(jnp.int32, (rr, 128), 1))
    iota_p = ((lax.broadcasted_iota(jnp.int32, (rp, 128), 0) % npb) * 128
              + lax.broadcasted_iota(jnp.int32, (rp, 128), 1))
    bsel = riota // nb
    bselp = riota_p // npb
    x, y, z = xc_ref[0], xc_ref[1], xc_ref[2]
    dists_ref[...] = jnp.full((rr, 128), 1e10, jnp.float32)
    idxs_ref[...] = jnp.zeros((rp, 128), jnp.int32)

    def _bcast(vals):
        acc = jnp.broadcast_to(vals[0], (riota.shape[0], 1))
        for bi in range(1, b):
            acc = jnp.where(bsel == bi, vals[bi], acc)
        return acc

    def body(i, carry):
        rows = [xr_ref[bi, pl.ds(carry[bi], 1), :] for bi in range(b)]
        cx = _bcast([r[:, 0:1] for r in rows])
        cy = _bcast([r[:, 1:2] for r in rows])
        cz = _bcast([r[:, 2:3] for r in rows])
        d = (x - cx) ** 2 + (y - cy) ** 2 + (z - cz) ** 2
        nd = jnp.minimum(dists_ref[...], d)
        dists_ref[...] = nd
        rm = jnp.max(nd, axis=1, keepdims=True)
        ms = [jnp.max(rm[bi * nb:(bi + 1) * nb]) for bi in range(b)]
        mcol = _bcast([m.reshape(1, 1) for m in ms])
        cand = jnp.where(nd == mcol, iota_n, n)
        nxts = [jnp.min(cand[bi * nb:(bi + 1) * nb]) for bi in range(b)]
        ncol = jnp.broadcast_to(nxts[0].reshape(1, 1), (rp, 1))
        for bi in range(1, b):
            ncol = jnp.where(bselp == bi, nxts[bi].reshape(1, 1), ncol)
        idxs_ref[...] = jnp.where(iota_p == i, ncol, idxs_ref[...])
        return tuple(nxts)

    lax.fori_loop(1, npoint, body, (jnp.int32(0),) * b)
    out_ref[...] = idxs_ref[...]


def _make_fps(b, n, npoint):
    nb, npb = n // 128, npoint // 128
    return pl.pallas_call(
        functools.partial(_fps_body, b, npoint, n),
        in_specs=[pl.BlockSpec(memory_space=pltpu.VMEM),
                  pl.BlockSpec(memory_space=pltpu.VMEM)],
        out_specs=pl.BlockSpec(memory_space=pltpu.VMEM),
        out_shape=jax.ShapeDtypeStruct((b * npb, 128), jnp.int32),
        scratch_shapes=[pltpu.VMEM((b * nb, 128), jnp.float32),
                        pltpu.VMEM((b * npb, 128), jnp.int32)],
    )


def _ballq_body(n, mc, ns, r2, cen_ref, ptT_ref, pk_ref, out_ref):
    nw = n // 16
    c = cen_ref[0]
    pt = ptT_ref[0]
    ab = jnp.dot(c, pt, preferred_element_type=jnp.float32)
    cn = jnp.sum(c * c, axis=1, keepdims=True)
    pn = jnp.sum(pt * pt, axis=0, keepdims=True)
    d2 = (cn + pn) - 2.0 * ab
    mask = jnp.where(d2 < r2, jnp.float32(1), jnp.float32(0)).astype(jnp.bfloat16)
    w = jnp.dot(mask, pk_ref[...], preferred_element_type=jnp.float32
                ).astype(jnp.int32)
    lane = lax.broadcasted_iota(jnp.int32, (mc, nw), 1)
    cols = lax.broadcasted_iota(jnp.int32, (mc, ns), 1)
    g = jnp.full((mc, ns), n, jnp.int32)
    for k in range(ns):
        wid = jnp.min(jnp.where(w > 0, lane, nw), axis=1, keepdims=True)
        hit = lane == wid
        wv = jnp.max(jnp.where(hit, w, 0), axis=1, keepdims=True)
        lsb = wv & (-wv)
        t = (lax.bitcast_convert_type(lsb.astype(jnp.float32), jnp.int32)
             >> 23) - 127
        idxk = jnp.where(wv == 0, n, wid * 16 + t)
        g = jnp.where(cols == k, idxk, g)
        w = jnp.where(hit, w & (w - 1), w)
    first = jnp.broadcast_to(g[:, 0:1], (mc, ns))
    g = jnp.where(g == n, first, g)
    g = jnp.where(g == n, 0, g)
    out_ref[0] = g


def _make_ballq(b, n, m, mc, ns, radius):
    r2 = float(np.float32(radius * radius))
    nw = n // 16
    call = pl.pallas_call(
        functools.partial(_ballq_body, n, mc, ns, r2),
        grid=(b, m // mc),
        in_specs=[pl.BlockSpec((1, mc, 8), lambda i, j: (i, j, 0)),
                  pl.BlockSpec((1, 8, n), lambda i, j: (i, 0, 0)),
                  pl.BlockSpec((n, nw), lambda i, j: (0, 0))],
        out_specs=pl.BlockSpec((1, mc, ns), lambda i, j: (i, j, 0)),
        out_shape=jax.ShapeDtypeStruct((b, m, ns), jnp.int32),
    )
    pknp = np.zeros((n, nw), np.float32)
    ar = np.arange(n)
    pknp[ar, ar // 16] = (2.0 ** (ar % 16)).astype(np.float32)
    pk = jnp.asarray(pknp, jnp.bfloat16)
    return lambda cen, ptT: call(cen, ptT, pk)


def _mlp_body(mc, ns, g_ref, c_ref, w1_ref, w2_ref, w3_ref, wa_ref, out_ref):
    din = g_ref.shape[2]
    g = g_ref[0].reshape(mc, ns, din)
    gc = (g - c_ref[0][:, None, :]).reshape(mc * ns, din)
    h1 = _relu(jnp.dot(gc, w1_ref[...], preferred_element_type=jnp.float32))
    h2 = _relu(jnp.dot(h1, w2_ref[...], preferred_element_type=jnp.float32))
    h3 = _relu(jnp.dot(h2, w3_ref[...], preferred_element_type=jnp.float32))
    d3 = h3.shape[1]
    mx = jnp.max(h3.reshape(mc, ns, d3), axis=1)
    out_ref[0] = _relu(jnp.dot(mx, wa_ref[...], preferred_element_type=jnp.float32))


def _make_mlp(b, m, mc, ns, din, d1, d2, d3, do):
    return pl.pallas_call(
        functools.partial(_mlp_body, mc, ns),
        grid=(b, m // mc),
        in_specs=[pl.BlockSpec((1, mc * ns, din), lambda i, j: (i, j, 0)),
                  pl.BlockSpec((1, mc, din), lambda i, j: (i, j, 0)),
                  pl.BlockSpec((din, d1), lambda i, j: (0, 0)),
                  pl.BlockSpec((d1, d2), lambda i, j: (0, 0)),
                  pl.BlockSpec((d2, d3), lambda i, j: (0, 0)),
                  pl.BlockSpec((d3, do), lambda i, j: (0, 0))],
        out_specs=pl.BlockSpec((1, mc, do), lambda i, j: (i, j, 0)),
        out_shape=jax.ShapeDtypeStruct((b, m, do), jnp.float32),
    )


def _seg_body(m, kk, f_ref, w1_ref, w2_ref, b2_ref, s_ref, fg_ref, key_ref):
    h = _relu(jnp.dot(f_ref[0], w1_ref[...], preferred_element_type=jnp.float32))
    s = jnp.dot(h, w2_ref[...], preferred_element_type=jnp.float32) + b2_ref[...]
    s_ref[0] = s
    mb = m // 128
    kb = kk // 128
    col = lax.broadcasted_iota(jnp.int32, (m, 8), 1)
    smax = jnp.max(jnp.where(col < 3, s, -1e30), axis=1)
    key_ref[...] = jax.nn.sigmoid(smax).reshape(mb, 128)
    iota_m = (lax.broadcasted_iota(jnp.int32, (mb, 128), 0) * 128
              + lax.broadcasted_iota(jnp.int32, (mb, 128), 1)).astype(jnp.float32)
    iota_k = (lax.broadcasted_iota(jnp.int32, (kb, 128), 0) * 128
              + lax.broadcasted_iota(jnp.int32, (kb, 128), 1)).astype(jnp.float32)
    fg_ref[0] = jnp.zeros((kb, 128), jnp.int32)

    def body(i, _):
        kv = key_ref[...]
        mx = jnp.max(kv)
        pos = jnp.min(jnp.where(kv == mx, iota_m, float(m)))
        fg_ref[0] = jnp.where(iota_k == i.astype(jnp.float32),
                              pos.astype(jnp.int32), fg_ref[0])
        key_ref[...] = jnp.where(iota_m == pos, -3e30, kv)
        return 0

    lax.fori_loop(0, kk, body, 0)


def _make_seg(b, m, kk):
    return pl.pallas_call(
        functools.partial(_seg_body, m, kk),
        grid=(b,),
        in_specs=[pl.BlockSpec((1, m, 128), lambda i: (i, 0, 0)),
                  pl.BlockSpec((128, 64), lambda i: (0, 0)),
                  pl.BlockSpec((64, 8), lambda i: (0, 0)),
                  pl.BlockSpec((1, 8), lambda i: (0, 0))],
        out_specs=[pl.BlockSpec((1, m, 8), lambda i: (i, 0, 0)),
                   pl.BlockSpec((1, kk // 128, 128), lambda i: (i, 0, 0))],
        out_shape=[jax.ShapeDtypeStruct((b, m, 8), jnp.float32),
                   jax.ShapeDtypeStruct((b, kk // 128, 128), jnp.int32)],
        scratch_shapes=[pltpu.VMEM((m // 128, 128), jnp.float32)],
    )


def _vote_body(x3_ref, f3_ref, wm_ref, rw_ref, rb_ref, mt_ref,
               off_ref, x4_ref):
    f3 = f3_ref[0]
    x3 = x3_ref[0]
    vh = _relu(jnp.dot(f3, wm_ref[...], preferred_element_type=jnp.float32))
    off = jnp.dot(vh, rw_ref[...], preferred_element_type=jnp.float32) + rb_ref[...]
    mt = mt_ref[...]
    lim = jnp.clip(off, -mt, mt)
    off_ref[0] = off
    x4_ref[0] = x3 + lim


def _make_vote(b, m):
    return pl.pallas_call(
        _vote_body,
        grid=(b,),
        in_specs=[pl.BlockSpec((1, m, 8), lambda i: (i, 0, 0)),
                  pl.BlockSpec((1, m, 128), lambda i: (i, 0, 0)),
                  pl.BlockSpec((128, 128), lambda i: (0, 0)),
                  pl.BlockSpec((128, 8), lambda i: (0, 0)),
                  pl.BlockSpec((1, 8), lambda i: (0, 0)),
                  pl.BlockSpec((1, 8), lambda i: (0, 0))],
        out_specs=[pl.BlockSpec((1, m, 8), lambda i: (i, 0, 0)),
                   pl.BlockSpec((1, m, 8), lambda i: (i, 0, 0))],
        out_shape=[jax.ShapeDtypeStruct((b, m, 8), jnp.float32),
                   jax.ShapeDtypeStruct((b, m, 8), jnp.float32)],
    )


def _make_gather(b, r, d, mtot):
    cpw = mtot // _NW
    sub = min(cpw, 128)
    nsub = cpw // sub
    wpb = _NW // b
    mesh = plsc.VectorSubcoreMesh(core_axis_name="c", subcore_axis_name="s")

    @functools.partial(
        pl.kernel,
        out_type=jax.ShapeDtypeStruct((mtot, d), jnp.float32),
        mesh=mesh,
        scratch_types=[pltpu.VMEM((sub,), jnp.int32),
                       pltpu.VMEM((sub, d), jnp.float32),
                       pltpu.SemaphoreType.DMA],
    )
    def gk(table_hbm, idx_hbm, out_hbm, idx_v, rows_v, sem):
        wid = lax.axis_index("s") * 2 + lax.axis_index("c")
        base0 = wid * cpw
        boff = (wid // wpb) * r

        def body(j, _):
            base = base0 + j * sub
            pltpu.sync_copy(idx_hbm.at[pl.ds(base, sub)], idx_v)
            for t in range(sub // 16):
                idx_v[pl.ds(t * 16, 16)] = idx_v[pl.ds(t * 16, 16)] + boff
            pltpu.async_copy(table_hbm.at[idx_v], rows_v, sem).wait()
            pltpu.sync_copy(rows_v, out_hbm.at[pl.ds(base, sub)])
            return 0

        lax.fori_loop(0, nsub, body, 0)

    return gk


def _pad3(x, w=8):
    return jnp.concatenate(
        [x, jnp.zeros(x.shape[:-1] + (w - x.shape[-1],), x.dtype)], axis=-1)


def kernel(points, params, batch_size):
    b = points.shape[0] // _N
    n = _N
    xyz = points[:, 1:4].reshape(b, n, 3)
    feat = points[:, 4:5].reshape(b, n, 1)
    bcol = points[:, 0].reshape(b, n)

    p0 = params["sa0"]
    p1 = params["sa1"]
    p4 = params["sa4"]
    w1a = jnp.zeros((8, 32), jnp.float32).at[0:4].set(p0["mlp"][0])
    w1b = jnp.zeros((72, 64), jnp.float32).at[0:67].set(p1["mlp"][0])
    w1c = jnp.zeros((136, 128), jnp.float32).at[0:131].set(p4["mlp"][0])
    segw2 = jnp.zeros((64, 8), jnp.float32).at[:, 0:3].set(params["seg"]["w2"])
    segb2 = jnp.zeros((1, 8), jnp.float32).at[0, 0:3].set(params["seg"]["b2"])
    regw = jnp.zeros((128, 8), jnp.float32).at[:, 0:3].set(params["vote"]["reg_w"])
    regb = jnp.zeros((1, 8), jnp.float32).at[0, 0:3].set(params["vote"]["reg_b"])
    mt = jnp.asarray(np.concatenate([_MAXT, np.full((5,), 1e30, np.float32)])[None])

    xg0 = xyz.transpose(0, 2, 1).reshape(b, 3, n // 128, 128)
    xc0 = jnp.concatenate([xg0[i] for i in range(b)], axis=1)
    xyz8 = _pad3(xyz)
    idx0 = _make_fps(b, n, 2048)(xc0, xyz8).reshape(b * 2048)
    xyzf128 = _pad3(jnp.concatenate([xyz, feat], axis=-1), 128).reshape(b * n, 128)
    cen0 = _make_gather(b, n, 128, b * 2048)(xyzf128, idx0)[:, 0:3].reshape(b, 2048, 3)
    cen0_8 = _pad3(cen0)
    ptT0 = _pad3(xyz).transpose(0, 2, 1)
    gidx0 = _make_ballq(b, n, 2048, 512, 32, 1.0)(cen0_8, ptT0)
    g0 = _make_gather(b, n, 128, b * 2048 * 32)(
        xyzf128, gidx0.reshape(-1))[:, 0:8].reshape(b, 2048 * 32, 8)
    f1 = _make_mlp(b, 2048, 128, 32, 8, 32, 32, 64, 64)(
        g0, cen0_8, w1a, p0["mlp"][1], p0["mlp"][2], p0["agg"])

    xg1 = cen0.transpose(0, 2, 1).reshape(b, 3, 16, 128)
    xc1 = jnp.concatenate([xg1[i] for i in range(b)], axis=1)
    idx1 = _make_fps(b, 2048, 512)(xc1, cen0_8).reshape(b * 512)
    in1_128 = _pad3(jnp.concatenate([cen0, f1], axis=-1), 128).reshape(b * 2048, 128)
    cen1 = _make_gather(b, 2048, 128, b * 512)(in1_128, idx1)[:, 0:3].reshape(b, 512, 3)
    cen1_8 = _pad3(cen1)
    ptT1 = _pad3(cen0).transpose(0, 2, 1)
    gidx1 = _make_ballq(b, 2048, 512, 512, 32, 2.0)(cen1_8, ptT1)
    g1 = _make_gather(b, 2048, 128, b * 512 * 32)(
        in1_128, gidx1.reshape(-1))[:, 0:72].reshape(b, 512 * 32, 72)
    cen1_72 = _pad3(cen1, 72)
    f2 = _make_mlp(b, 512, 128, 32, 72, 64, 64, 128, 128)(
        g1, cen1_72, w1b, p1["mlp"][1], p1["mlp"][2], p1["agg"])

    segs, fg = _make_seg(b, 512, 256)(f2, params["seg"]["w1"], segw2, segb2)
    fg_idx = fg.reshape(b * 256)
    t3 = jnp.concatenate([cen1, f2, jnp.zeros((b, 512, 125), jnp.float32)],
                         axis=-1).reshape(b * 512, 256)
    sel = _make_gather(b, 512, 256, b * 256)(t3, fg_idx)
    x3 = sel[:, 0:3].reshape(b, 256, 3)
    f3 = sel[:, 3:131].reshape(b, 256, 128)

    x3_8 = _pad3(x3)
    off8, x4_8 = _make_vote(b, 256)(
        x3_8, f3, params["vote"]["mlp"], regw, regb, mt)
    ptT4 = _pad3(x3).transpose(0, 2, 1)
    gidx4 = _make_ballq(b, 256, 256, 256, 32, 4.8)(x4_8, ptT4)
    t4 = jnp.concatenate([x3, f3, jnp.zeros((b, 256, 125), jnp.float32)],
                         axis=-1).reshape(b * 256, 256)
    g4 = _make_gather(b, 256, 256, b * 256 * 32)(
        t4, gidx4.reshape(-1))[:, 0:136].reshape(b, 256 * 32, 136)
    x4_136 = _pad3(x4_8[..., 0:3], 136)
    f5 = _make_mlp(b, 256, 256, 32, 136, 128, 128, 256, 256)(
        g4, x4_136, w1c, p4["mlp"][1], p4["mlp"][2], p4["agg"])

    ctr_bidx = bcol[:, :256].reshape(-1, 1)
    ctr_offsets_out = jnp.concatenate([ctr_bidx, off8[..., 0:3].reshape(-1, 3)], axis=1)
    centers_out = jnp.concatenate([ctr_bidx, x4_8[..., 0:3].reshape(-1, 3)], axis=1)
    centers_origin_out = jnp.concatenate([ctr_bidx, x3.reshape(-1, 3)], axis=1)
    centers_features = f5.reshape(-1, 256)
    seg_preds = jnp.concatenate([bcol[:, :512][..., None], segs[..., 0:3]], axis=-1)
    seg_points = jnp.concatenate([bcol[:, :512].reshape(-1, 1),
                                  cen1.reshape(-1, 3)], axis=1)
    return (ctr_offsets_out, centers_out, centers_origin_out, centers_features,
            seg_preds, seg_points)


# speedup vs baseline: 1.6584x; 1.0091x over previous
import functools

import jax
import jax.numpy as jnp
import numpy as np
from jax import lax
from jax.experimental import pallas as pl
from jax.experimental.pallas import tpu as pltpu
from jax.experimental.pallas import tpu_sc as plsc

_N = 8192
_MAXT = np.array([3.0, 3.0, 2.0], dtype=np.float32)
_NW = 32


def _relu(x):
    return jnp.maximum(x, 0.0)


def _fps_body(b, npoint, n, xc_ref, xr_ref, out_ref, dists_ref, idxs_ref):
    nb = n // 128
    npb = npoint // 128
    rr, rp = b * nb, b * npb
    riota = lax.broadcasted_iota(jnp.int32, (rr, 1), 0)
    riota_p = lax.broadcasted_iota(jnp.int32, (rp, 1), 0)
    iota_n = ((lax.broadcasted_iota(jnp.int32, (rr, 128), 0) % nb) * 128
              + lax.broadcasted_iota(jnp.int32, (rr, 128), 1))
    iota_p = ((lax.broadcasted_iota(jnp.int32, (rp, 128), 0) % npb) * 128
              + lax.broadcasted_iota(jnp.int32, (rp, 128), 1))
    bsel = riota // nb
    bselp = riota_p // npb
    x, y, z = xc_ref[0], xc_ref[1], xc_ref[2]
    dists_ref[...] = jnp.full((rr, 128), 1e10, jnp.float32)
    idxs_ref[...] = jnp.zeros((rp, 128), jnp.int32)

    def _bcast(vals):
        acc = jnp.broadcast_to(vals[0], (riota.shape[0], 1))
        for bi in range(1, b):
            acc = jnp.where(bsel == bi, vals[bi], acc)
        return acc

    def body(i, carry):
        rows = [xr_ref[bi, pl.ds(carry[bi], 1), :] for bi in range(b)]
        cx = _bcast([r[:, 0:1] for r in rows])
        cy = _bcast([r[:, 1:2] for r in rows])
        cz = _bcast([r[:, 2:3] for r in rows])
        d = (x - cx) ** 2 + (y - cy) ** 2 + (z - cz) ** 2
        nd = jnp.minimum(dists_ref[...], d)
        dists_ref[...] = nd
        rm = jnp.max(nd, axis=1, keepdims=True)
        ms = [jnp.max(rm[bi * nb:(bi + 1) * nb]) for bi in range(b)]
        mcol = _bcast([m.reshape(1, 1) for m in ms])
        cand = jnp.where(nd == mcol, iota_n, n)
        nxts = [jnp.min(cand[bi * nb:(bi + 1) * nb]) for bi in range(b)]
        ncol = jnp.broadcast_to(nxts[0].reshape(1, 1), (rp, 1))
        for bi in range(1, b):
            ncol = jnp.where(bselp == bi, nxts[bi].reshape(1, 1), ncol)
        idxs_ref[...] = jnp.where(iota_p == i, ncol, idxs_ref[...])
        return tuple(nxts)

    lax.fori_loop(1, npoint, body, (jnp.int32(0),) * b)
    out_ref[...] = idxs_ref[...]


def _make_fps(b, n, npoint):
    nb, npb = n // 128, npoint // 128
    return pl.pallas_call(
        functools.partial(_fps_body, b, npoint, n),
        in_specs=[pl.BlockSpec(memory_space=pltpu.VMEM),
                  pl.BlockSpec(memory_space=pltpu.VMEM)],
        out_specs=pl.BlockSpec(memory_space=pltpu.VMEM),
        out_shape=jax.ShapeDtypeStruct((b * npb, 128), jnp.int32),
        scratch_shapes=[pltpu.VMEM((b * nb, 128), jnp.float32),
                        pltpu.VMEM((b * npb, 128), jnp.int32)],
    )


def _ballq_body(n, mc, ns, r2, cen_ref, ptT_ref, pk_ref, out_ref):
    nw = n // 16
    c = cen_ref[0]
    pt = ptT_ref[0]
    ab = jnp.dot(c, pt, preferred_element_type=jnp.float32)
    cn = jnp.sum(c * c, axis=1, keepdims=True)
    pn = jnp.sum(pt * pt, axis=0, keepdims=True)
    d2 = (cn + pn) - 2.0 * ab
    mask = jnp.where(d2 < r2, jnp.float32(1), jnp.float32(0)).astype(jnp.bfloat16)
    w = jnp.dot(mask, pk_ref[...], preferred_element_type=jnp.float32
                ).astype(jnp.int32)
    lane = lax.broadcasted_iota(jnp.int32, (mc, nw), 1)
    cols = lax.broadcasted_iota(jnp.int32, (mc, ns), 1)
    g = jnp.full((mc, ns), n, jnp.int32)
    for k in range(ns):
        wid = jnp.min(jnp.where(w > 0, lane, nw), axis=1, keepdims=True)
        hit = lane == wid
        wv = jnp.max(jnp.where(hit, w, 0), axis=1, keepdims=True)
        lsb = wv & (-wv)
        t = (lax.bitcast_convert_type(lsb.astype(jnp.float32), jnp.int32)
             >> 23) - 127
        idxk = jnp.where(wv == 0, n, wid * 16 + t)
        g = jnp.where(cols == k, idxk, g)
        w = jnp.where(hit, w & (w - 1), w)
    first = jnp.broadcast_to(g[:, 0:1], (mc, ns))
    g = jnp.where(g == n, first, g)
    g = jnp.where(g == n, 0, g)
    out_ref[0] = g


def _make_ballq(b, n, m, mc, ns, radius):
    r2 = float(np.float32(radius * radius))
    nw = n // 16
    call = pl.pallas_call(
        functools.partial(_ballq_body, n, mc, ns, r2),
        grid=(b, m // mc),
        in_specs=[pl.BlockSpec((1, mc, 8), lambda i, j: (i, j, 0)),
                  pl.BlockSpec((1, 8, n), lambda i, j: (i, 0, 0)),
                  pl.BlockSpec((n, nw), lambda i, j: (0, 0))],
        out_specs=pl.BlockSpec((1, mc, ns), lambda i, j: (i, j, 0)),
        out_shape=jax.ShapeDtypeStruct((b, m, ns), jnp.int32),
    )
    pknp = np.zeros((n, nw), np.float32)
    ar = np.arange(n)
    pknp[ar, ar // 16] = (2.0 ** (ar % 16)).astype(np.float32)
    pk = jnp.asarray(pknp, jnp.bfloat16)
    return lambda cen, ptT: call(cen, ptT, pk)


def _mlp_body(mc, ns, g_ref, c_ref, w1_ref, w2_ref, w3_ref, wa_ref, out_ref):
    din = g_ref.shape[2]
    g = g_ref[0].reshape(mc, ns, din)
    gc = (g - c_ref[0][:, None, :]).reshape(mc * ns, din)
    h1 = _relu(jnp.dot(gc, w1_ref[...], preferred_element_type=jnp.float32))
    h2 = _relu(jnp.dot(h1, w2_ref[...], preferred_element_type=jnp.float32))
    h3 = _relu(jnp.dot(h2, w3_ref[...], preferred_element_type=jnp.float32))
    d3 = h3.shape[1]
    mx = jnp.max(h3.reshape(mc, ns, d3), axis=1)
    out_ref[0] = _relu(jnp.dot(mx, wa_ref[...], preferred_element_type=jnp.float32))


def _make_mlp(b, m, mc, ns, din, d1, d2, d3, do):
    return pl.pallas_call(
        functools.partial(_mlp_body, mc, ns),
        grid=(b, m // mc),
        in_specs=[pl.BlockSpec((1, mc * ns, din), lambda i, j: (i, j, 0)),
                  pl.BlockSpec((1, mc, din), lambda i, j: (i, j, 0)),
                  pl.BlockSpec((din, d1), lambda i, j: (0, 0)),
                  pl.BlockSpec((d1, d2), lambda i, j: (0, 0)),
                  pl.BlockSpec((d2, d3), lambda i, j: (0, 0)),
                  pl.BlockSpec((d3, do), lambda i, j: (0, 0))],
        out_specs=pl.BlockSpec((1, mc, do), lambda i, j: (i, j, 0)),
        out_shape=jax.ShapeDtypeStruct((b, m, do), jnp.float32),
    )


def _seg_body(m, kk, f_ref, w1_ref, w2_ref, b2_ref, s_ref, fg_ref, key_ref):
    h = _relu(jnp.dot(f_ref[0], w1_ref[...], preferred_element_type=jnp.float32))
    s = jnp.dot(h, w2_ref[...], preferred_element_type=jnp.float32) + b2_ref[...]
    s_ref[0] = s
    mb = m // 128
    kb = kk // 128
    col = lax.broadcasted_iota(jnp.int32, (m, 8), 1)
    smax = jnp.max(jnp.where(col < 3, s, -1e30), axis=1)
    key_ref[...] = jax.nn.sigmoid(smax).reshape(mb, 128)
    iota_m = (lax.broadcasted_iota(jnp.int32, (mb, 128), 0) * 128
              + lax.broadcasted_iota(jnp.int32, (mb, 128), 1)).astype(jnp.float32)
    iota_k = (lax.broadcasted_iota(jnp.int32, (kb, 128), 0) * 128
              + lax.broadcasted_iota(jnp.int32, (kb, 128), 1)).astype(jnp.float32)
    fg_ref[0] = jnp.zeros((kb, 128), jnp.int32)

    def body(i, _):
        kv = key_ref[...]
        mx = jnp.max(kv)
        pos = jnp.min(jnp.where(kv == mx, iota_m, float(m)))
        fg_ref[0] = jnp.where(iota_k == i.astype(jnp.float32),
                              pos.astype(jnp.int32), fg_ref[0])
        key_ref[...] = jnp.where(iota_m == pos, -3e30, kv)
        return 0

    lax.fori_loop(0, kk, body, 0)


def _make_seg(b, m, kk):
    return pl.pallas_call(
        functools.partial(_seg_body, m, kk),
        grid=(b,),
        in_specs=[pl.BlockSpec((1, m, 128), lambda i: (i, 0, 0)),
                  pl.BlockSpec((128, 64), lambda i: (0, 0)),
                  pl.BlockSpec((64, 8), lambda i: (0, 0)),
                  pl.BlockSpec((1, 8), lambda i: (0, 0))],
        out_specs=[pl.BlockSpec((1, m, 8), lambda i: (i, 0, 0)),
                   pl.BlockSpec((1, kk // 128, 128), lambda i: (i, 0, 0))],
        out_shape=[jax.ShapeDtypeStruct((b, m, 8), jnp.float32),
                   jax.ShapeDtypeStruct((b, kk // 128, 128), jnp.int32)],
        scratch_shapes=[pltpu.VMEM((m // 128, 128), jnp.float32)],
    )


def _vote_body(x3_ref, f3_ref, wm_ref, rw_ref, rb_ref, mt_ref,
               off_ref, x4_ref):
    f3 = f3_ref[0]
    x3 = x3_ref[0]
    vh = _relu(jnp.dot(f3, wm_ref[...], preferred_element_type=jnp.float32))
    off = jnp.dot(vh, rw_ref[...], preferred_element_type=jnp.float32) + rb_ref[...]
    mt = mt_ref[...]
    lim = jnp.clip(off, -mt, mt)
    off_ref[0] = off
    x4_ref[0] = x3 + lim


def _make_vote(b, m):
    return pl.pallas_call(
        _vote_body,
        grid=(b,),
        in_specs=[pl.BlockSpec((1, m, 8), lambda i: (i, 0, 0)),
                  pl.BlockSpec((1, m, 128), lambda i: (i, 0, 0)),
                  pl.BlockSpec((128, 128), lambda i: (0, 0)),
                  pl.BlockSpec((128, 8), lambda i: (0, 0)),
                  pl.BlockSpec((1, 8), lambda i: (0, 0)),
                  pl.BlockSpec((1, 8), lambda i: (0, 0))],
        out_specs=[pl.BlockSpec((1, m, 8), lambda i: (i, 0, 0)),
                   pl.BlockSpec((1, m, 8), lambda i: (i, 0, 0))],
        out_shape=[jax.ShapeDtypeStruct((b, m, 8), jnp.float32),
                   jax.ShapeDtypeStruct((b, m, 8), jnp.float32)],
    )


def _make_gather(b, r, d, mtot):
    cpw = mtot // _NW
    sub = min(cpw, 128)
    nsub = cpw // sub
    kbuf = 1
    for cand in (4, 2):
        if nsub % cand == 0 and cand * sub * d <= 98304:
            kbuf = cand
            break
    ngrp = nsub // kbuf
    wpb = _NW // b
    mesh = plsc.VectorSubcoreMesh(core_axis_name="c", subcore_axis_name="s")

    @functools.partial(
        pl.kernel,
        out_type=jax.ShapeDtypeStruct((mtot, d), jnp.float32),
        mesh=mesh,
        scratch_types=([pltpu.VMEM((sub,), jnp.int32)] * kbuf
                       + [pltpu.VMEM((sub, d), jnp.float32)] * kbuf
                       + [pltpu.SemaphoreType.DMA]),
    )
    def gk(table_hbm, idx_hbm, out_hbm, *bufs):
        idxs = bufs[:kbuf]
        rows = bufs[kbuf:2 * kbuf]
        sem = bufs[2 * kbuf]
        wid = lax.axis_index("s") * 2 + lax.axis_index("c")
        base0 = wid * cpw
        boff = (wid // wpb) * r

        def body(g, _):
            for p in range(kbuf):
                base = base0 + (g * kbuf + p) * sub
                pltpu.sync_copy(idx_hbm.at[pl.ds(base, sub)], idxs[p])
                for t in range(sub // 16):
                    idxs[p][pl.ds(t * 16, 16)] = idxs[p][pl.ds(t * 16, 16)] + boff
            descs = [pltpu.async_copy(table_hbm.at[idxs[p]], rows[p], sem)
                     for p in range(kbuf)]
            for p in range(kbuf):
                descs[p].wait()
                base = base0 + (g * kbuf + p) * sub
                pltpu.sync_copy(rows[p], out_hbm.at[pl.ds(base, sub)])
            return 0

        lax.fori_loop(0, ngrp, body, 0)

    return gk


def _pad3(x, w=8):
    return jnp.concatenate(
        [x, jnp.zeros(x.shape[:-1] + (w - x.shape[-1],), x.dtype)], axis=-1)


def kernel(points, params, batch_size):
    b = points.shape[0] // _N
    n = _N
    xyz = points[:, 1:4].reshape(b, n, 3)
    feat = points[:, 4:5].reshape(b, n, 1)
    bcol = points[:, 0].reshape(b, n)

    p0 = params["sa0"]
    p1 = params["sa1"]
    p4 = params["sa4"]
    w1a = jnp.zeros((8, 32), jnp.float32).at[0:4].set(p0["mlp"][0])
    w1b = jnp.zeros((72, 64), jnp.float32).at[0:67].set(p1["mlp"][0])
    w1c = jnp.zeros((136, 128), jnp.float32).at[0:131].set(p4["mlp"][0])
    segw2 = jnp.zeros((64, 8), jnp.float32).at[:, 0:3].set(params["seg"]["w2"])
    segb2 = jnp.zeros((1, 8), jnp.float32).at[0, 0:3].set(params["seg"]["b2"])
    regw = jnp.zeros((128, 8), jnp.float32).at[:, 0:3].set(params["vote"]["reg_w"])
    regb = jnp.zeros((1, 8), jnp.float32).at[0, 0:3].set(params["vote"]["reg_b"])
    mt = jnp.asarray(np.concatenate([_MAXT, np.full((5,), 1e30, np.float32)])[None])

    xg0 = xyz.transpose(0, 2, 1).reshape(b, 3, n // 128, 128)
    xc0 = jnp.concatenate([xg0[i] for i in range(b)], axis=1)
    xyz8 = _pad3(xyz)
    idx0 = _make_fps(b, n, 2048)(xc0, xyz8).reshape(b * 2048)
    xyzf128 = _pad3(jnp.concatenate([xyz, feat], axis=-1), 128).reshape(b * n, 128)
    cen0 = _make_gather(b, n, 128, b * 2048)(xyzf128, idx0)[:, 0:3].reshape(b, 2048, 3)
    cen0_8 = _pad3(cen0)
    ptT0 = _pad3(xyz).transpose(0, 2, 1)
    gidx0 = _make_ballq(b, n, 2048, 512, 32, 1.0)(cen0_8, ptT0)
    g0 = _make_gather(b, n, 128, b * 2048 * 32)(
        xyzf128, gidx0.reshape(-1))[:, 0:8].reshape(b, 2048 * 32, 8)
    f1 = _make_mlp(b, 2048, 128, 32, 8, 32, 32, 64, 64)(
        g0, cen0_8, w1a, p0["mlp"][1], p0["mlp"][2], p0["agg"])

    xg1 = cen0.transpose(0, 2, 1).reshape(b, 3, 16, 128)
    xc1 = jnp.concatenate([xg1[i] for i in range(b)], axis=1)
    idx1 = _make_fps(b, 2048, 512)(xc1, cen0_8).reshape(b * 512)
    in1_128 = _pad3(jnp.concatenate([cen0, f1], axis=-1), 128).reshape(b * 2048, 128)
    cen1 = _make_gather(b, 2048, 128, b * 512)(in1_128, idx1)[:, 0:3].reshape(b, 512, 3)
    cen1_8 = _pad3(cen1)
    ptT1 = _pad3(cen0).transpose(0, 2, 1)
    gidx1 = _make_ballq(b, 2048, 512, 512, 32, 2.0)(cen1_8, ptT1)
    g1 = _make_gather(b, 2048, 128, b * 512 * 32)(
        in1_128, gidx1.reshape(-1))[:, 0:72].reshape(b, 512 * 32, 72)
    cen1_72 = _pad3(cen1, 72)
    f2 = _make_mlp(b, 512, 128, 32, 72, 64, 64, 128, 128)(
        g1, cen1_72, w1b, p1["mlp"][1], p1["mlp"][2], p1["agg"])

    segs, fg = _make_seg(b, 512, 256)(f2, params["seg"]["w1"], segw2, segb2)
    fg_idx = fg.reshape(b * 256)
    t3 = jnp.concatenate([cen1, f2, jnp.zeros((b, 512, 125), jnp.float32)],
                         axis=-1).reshape(b * 512, 256)
    sel = _make_gather(b, 512, 256, b * 256)(t3, fg_idx)
    x3 = sel[:, 0:3].reshape(b, 256, 3)
    f3 = sel[:, 3:131].reshape(b, 256, 128)

    x3_8 = _pad3(x3)
    off8, x4_8 = _make_vote(b, 256)(
        x3_8, f3, params["vote"]["mlp"], regw, regb, mt)
    ptT4 = _pad3(x3).transpose(0, 2, 1)
    gidx4 = _make_ballq(b, 256, 256, 256, 32, 4.8)(x4_8, ptT4)
    t4 = jnp.concatenate([x3, f3, jnp.zeros((b, 256, 125), jnp.float32)],
                         axis=-1).reshape(b * 256, 256)
    g4 = _make_gather(b, 256, 256, b * 256 * 32)(
        t4, gidx4.reshape(-1))[:, 0:136].reshape(b, 256 * 32, 136)
    x4_136 = _pad3(x4_8[..., 0:3], 136)
    f5 = _make_mlp(b, 256, 256, 32, 136, 128, 128, 256, 256)(
        g4, x4_136, w1c, p4["mlp"][1], p4["mlp"][2], p4["agg"])

    ctr_bidx = bcol[:, :256].reshape(-1, 1)
    ctr_offsets_out = jnp.concatenate([ctr_bidx, off8[..., 0:3].reshape(-1, 3)], axis=1)
    centers_out = jnp.concatenate([ctr_bidx, x4_8[..., 0:3].reshape(-1, 3)], axis=1)
    centers_origin_out = jnp.concatenate([ctr_bidx, x3.reshape(-1, 3)], axis=1)
    centers_features = f5.reshape(-1, 256)
    seg_preds = jnp.concatenate([bcol[:, :512][..., None], segs[..., 0:3]], axis=-1)
    seg_points = jnp.concatenate([bcol[:, :512].reshape(-1, 1),
                                  cen1.reshape(-1, 3)], axis=1)
    return (ctr_offsets_out, centers_out, centers_origin_out, centers_features,
            seg_preds, seg_points)
